# baseline (speedup 1.0000x reference)
import functools

import jax
import jax.numpy as jnp
from jax import lax
from jax.experimental import pallas as pl
from jax.experimental.pallas import tpu as pltpu
from jax.experimental.pallas import tpu_sc as plsc

NC = 2
NS = 16
NW = NC * NS
LANES = 16


def _sc_kernel(B, F, CARD2, E):
    rows_per_w = B // NW
    n_slices = rows_per_w // 128
    mesh = plsc.VectorSubcoreMesh(core_axis_name="c", subcore_axis_name="s",
                                  num_cores=NC, num_subcores=NS)

    @functools.partial(
        pl.kernel,
        out_type=jax.ShapeDtypeStruct((B, E), jnp.float32),
        mesh=mesh,
        compiler_params=pltpu.CompilerParams(use_tc_tiling_on_sc=False),
        scratch_types=[
            pltpu.VMEM((F, n_slices, 128), jnp.int32),
            pltpu.VMEM((n_slices, 128), jnp.int32),
            pltpu.VMEM_SHARED((NS * rows_per_w, E), jnp.float32),
            pltpu.VMEM((128, E), jnp.float32),
            pltpu.SemaphoreType.DMA,
            pltpu.SemaphoreType.DMA,
        ],
    )
    def kern(tab_hbm, idx_hbm, out_hbm, idx_v, oidx_v, acc_sh, buf_v,
             gsem, ssem):
        sid = lax.axis_index("s")
        wid = sid * NC + lax.axis_index("c")
        base = sid * rows_per_w
        pltpu.sync_copy(idx_hbm.at[wid], idx_v)

        zeros16 = jnp.zeros((LANES,), jnp.float32)

        @pl.loop(0, n_slices)
        def _(m):
            @pl.loop(0, 128 // LANES)
            def _(k):
                oidx_v[m, pl.ds(k * LANES, LANES)] = (
                    lax.iota(jnp.int32, LANES) + (base + m * 128 + k * LANES))

        @pl.loop(0, 128)
        def _(r):
            buf_v[r, pl.ds(0, LANES)] = zeros16
            buf_v[r, pl.ds(LANES, LANES)] = zeros16
        for m in range(n_slices):
            pltpu.sync_copy(buf_v, acc_sh.at[pl.ds(base + m * 128, 128)])

        @pl.loop(0, F)
        def _(f):
            for m in range(n_slices):
                pltpu.async_copy(tab_hbm.at[idx_v.at[f, m]], buf_v, gsem).wait()
                pltpu.async_copy(buf_v, acc_sh.at[oidx_v.at[m]], ssem,
                                 add=True).wait()

        pltpu.sync_copy(acc_sh.at[pl.ds(base, rows_per_w)],
                        out_hbm.at[pl.ds(wid * rows_per_w, rows_per_w)])

    return kern


def kernel(indices, tables):
    F, CARD2, E = tables.shape
    B = indices.shape[0]
    tab_flat = tables.reshape(F * CARD2, E)
    offs = jnp.arange(F, dtype=jnp.int32) * CARD2 + 1
    flat = (indices.astype(jnp.int32) + offs[None, :]).T
    rows_per_w = B // NW
    idx_arr = (flat.reshape(F, NW, rows_per_w)
                   .transpose(1, 0, 2)
                   .reshape(NW, F, rows_per_w // 128, 128))
    return _sc_kernel(B, F, CARD2, E)(tab_flat, idx_arr)
